# baseline (device time: 105541 ns/iter reference)
import jax
import jax.numpy as jnp
from jax import lax
from jax.experimental import pallas as pl
from jax.experimental.pallas import tpu as pltpu

N_DEV = 4
N_STEP = 2 * (N_DEV - 1)
K_DMA = 16


def _body(idx_ref, mask_ref, e_ref, out_ref, cw_buf, ccw_buf,
          gather_sems, cw_ssem, cw_rsem, ccw_ssem, ccw_rsem):
    t, d = out_ref.shape
    half = t // 2
    chunk = half // N_DEV

    my = lax.axis_index("i")
    left = (my - 1) % N_DEV
    right = (my + 1) % N_DEV

    def cw_rows(c):
        return pl.ds(c * chunk, chunk)

    def ccw_rows(c):
        return pl.ds(half + c * chunk, chunk)

    def row_dma(i):
        return pltpu.make_async_copy(
            e_ref.at[pl.ds(idx_ref[i], 1), :],
            out_ref.at[pl.ds(i, 1), :],
            gather_sems.at[i % K_DMA],
        )

    barrier_sem = pltpu.get_barrier_semaphore()
    for nbr in [left, right]:
        pl.semaphore_signal(
            barrier_sem, inc=1,
            device_id=(nbr,), device_id_type=pl.DeviceIdType.MESH,
        )
    pl.semaphore_wait(barrier_sem, 2)

    def gather_step(i, carry):
        @pl.when(i >= K_DMA)
        def _():
            row_dma(i - K_DMA).wait()
        row_dma(i).start()
        return carry

    lax.fori_loop(0, t, gather_step, 0)

    def drain_step(i, carry):
        row_dma(i).wait()
        return carry

    lax.fori_loop(t - K_DMA, t, drain_step, 0)

    out_ref[:, :] = out_ref[:, :] * mask_ref[:, :]

    for s in range(N_DEV - 1):
        cw_sc = (my - s) % N_DEV
        cw_rc = (my - s - 1) % N_DEV
        ccw_sc = (my + s) % N_DEV
        ccw_rc = (my + s + 1) % N_DEV
        cw = pltpu.make_async_remote_copy(
            src_ref=out_ref.at[cw_rows(cw_sc), :],
            dst_ref=cw_buf.at[s],
            send_sem=cw_ssem.at[s],
            recv_sem=cw_rsem.at[s],
            device_id=(right,),
            device_id_type=pl.DeviceIdType.MESH,
        )
        ccw = pltpu.make_async_remote_copy(
            src_ref=out_ref.at[ccw_rows(ccw_sc), :],
            dst_ref=ccw_buf.at[s],
            send_sem=ccw_ssem.at[s],
            recv_sem=ccw_rsem.at[s],
            device_id=(left,),
            device_id_type=pl.DeviceIdType.MESH,
        )
        cw.start()
        ccw.start()
        cw.wait()
        ccw.wait()
        out_ref[cw_rows(cw_rc), :] = (
            out_ref[cw_rows(cw_rc), :] + cw_buf[s, :, :]
        )
        out_ref[ccw_rows(ccw_rc), :] = (
            out_ref[ccw_rows(ccw_rc), :] + ccw_buf[s, :, :]
        )

    for s in range(N_DEV - 1):
        k = (N_DEV - 1) + s
        cw_sc = (my + 1 - s) % N_DEV
        ccw_sc = (my - 1 + s) % N_DEV
        cw = pltpu.make_async_remote_copy(
            src_ref=out_ref.at[cw_rows(cw_sc), :],
            dst_ref=out_ref.at[cw_rows(cw_sc), :],
            send_sem=cw_ssem.at[k],
            recv_sem=cw_rsem.at[k],
            device_id=(right,),
            device_id_type=pl.DeviceIdType.MESH,
        )
        ccw = pltpu.make_async_remote_copy(
            src_ref=out_ref.at[ccw_rows(ccw_sc), :],
            dst_ref=out_ref.at[ccw_rows(ccw_sc), :],
            send_sem=ccw_ssem.at[k],
            recv_sem=ccw_rsem.at[k],
            device_id=(left,),
            device_id_type=pl.DeviceIdType.MESH,
        )
        cw.start()
        ccw.start()
        cw.wait()
        ccw.wait()


def kernel(ids, E):
    v_per, d = E.shape
    t = ids.shape[0]
    my_pos = lax.axis_index("i")

    local = ids - my_pos * v_per
    mask = (local >= 0) & (local < v_per)
    idx = jnp.clip(local, 0, v_per - 1).astype(jnp.int32)
    maskf = mask.astype(jnp.float32)[:, None]

    chunk = t // (2 * N_DEV)
    return pl.pallas_call(
        _body,
        out_shape=jax.ShapeDtypeStruct((t, d), jnp.float32),
        in_specs=[
            pl.BlockSpec(memory_space=pltpu.SMEM),
            pl.BlockSpec(memory_space=pltpu.VMEM),
            pl.BlockSpec(memory_space=pltpu.MemorySpace.HBM),
        ],
        out_specs=pl.BlockSpec(memory_space=pltpu.VMEM),
        scratch_shapes=[
            pltpu.VMEM((N_DEV - 1, chunk, d), jnp.float32),
            pltpu.VMEM((N_DEV - 1, chunk, d), jnp.float32),
            pltpu.SemaphoreType.DMA((K_DMA,)),
            pltpu.SemaphoreType.DMA((N_STEP,)),
            pltpu.SemaphoreType.DMA((N_STEP,)),
            pltpu.SemaphoreType.DMA((N_STEP,)),
            pltpu.SemaphoreType.DMA((N_STEP,)),
        ],
        compiler_params=pltpu.CompilerParams(collective_id=0),
    )(idx, maskf, E)


# device time: 86400 ns/iter; 1.2215x vs baseline; 1.2215x over previous
import jax
import jax.numpy as jnp
from jax import lax
from jax.experimental import pallas as pl
from jax.experimental.pallas import tpu as pltpu

N_DEV = 4
N_STEP = 2 * (N_DEV - 1)
K_DMA = 32


def _body(idx_ref, hit_ref, e_ref, out_ref, cw_buf, ccw_buf,
          gather_sems, cw_ssem, cw_rsem, ccw_ssem, ccw_rsem):
    t, d = out_ref.shape
    half = t // 2
    chunk = half // N_DEV

    my = lax.axis_index("i")
    left = (my - 1) % N_DEV
    right = (my + 1) % N_DEV

    def cw_rows(c):
        return pl.ds(c * chunk, chunk)

    def ccw_rows(c):
        return pl.ds(half + c * chunk, chunk)

    def row_dma(i):
        return pltpu.make_async_copy(
            e_ref.at[pl.ds(idx_ref[i], 1), :],
            out_ref.at[pl.ds(i, 1), :],
            gather_sems.at[i % K_DMA],
        )

    barrier_sem = pltpu.get_barrier_semaphore()
    for nbr in [left, right]:
        pl.semaphore_signal(
            barrier_sem, inc=1,
            device_id=(nbr,), device_id_type=pl.DeviceIdType.MESH,
        )
    pl.semaphore_wait(barrier_sem, 2)

    out_ref[:, :] = jnp.zeros((t, d), jnp.float32)

    def gather_step(i, carry):
        @pl.when((i >= K_DMA) & (hit_ref[i - K_DMA] == 1))
        def _():
            row_dma(i - K_DMA).wait()

        @pl.when(hit_ref[i] == 1)
        def _():
            row_dma(i).start()

        return carry

    lax.fori_loop(0, t, gather_step, 0)

    def drain_step(i, carry):
        @pl.when(hit_ref[i] == 1)
        def _():
            row_dma(i).wait()
        return carry

    lax.fori_loop(t - K_DMA, t, drain_step, 0)

    for s in range(N_DEV - 1):
        cw_sc = (my - s) % N_DEV
        cw_rc = (my - s - 1) % N_DEV
        ccw_sc = (my + s) % N_DEV
        ccw_rc = (my + s + 1) % N_DEV
        cw = pltpu.make_async_remote_copy(
            src_ref=out_ref.at[cw_rows(cw_sc), :],
            dst_ref=cw_buf.at[s],
            send_sem=cw_ssem.at[s],
            recv_sem=cw_rsem.at[s],
            device_id=(right,),
            device_id_type=pl.DeviceIdType.MESH,
        )
        ccw = pltpu.make_async_remote_copy(
            src_ref=out_ref.at[ccw_rows(ccw_sc), :],
            dst_ref=ccw_buf.at[s],
            send_sem=ccw_ssem.at[s],
            recv_sem=ccw_rsem.at[s],
            device_id=(left,),
            device_id_type=pl.DeviceIdType.MESH,
        )
        cw.start()
        ccw.start()
        cw.wait()
        ccw.wait()
        out_ref[cw_rows(cw_rc), :] = (
            out_ref[cw_rows(cw_rc), :] + cw_buf[s, :, :]
        )
        out_ref[ccw_rows(ccw_rc), :] = (
            out_ref[ccw_rows(ccw_rc), :] + ccw_buf[s, :, :]
        )

    for s in range(N_DEV - 1):
        k = (N_DEV - 1) + s
        cw_sc = (my + 1 - s) % N_DEV
        ccw_sc = (my - 1 + s) % N_DEV
        cw = pltpu.make_async_remote_copy(
            src_ref=out_ref.at[cw_rows(cw_sc), :],
            dst_ref=out_ref.at[cw_rows(cw_sc), :],
            send_sem=cw_ssem.at[k],
            recv_sem=cw_rsem.at[k],
            device_id=(right,),
            device_id_type=pl.DeviceIdType.MESH,
        )
        ccw = pltpu.make_async_remote_copy(
            src_ref=out_ref.at[ccw_rows(ccw_sc), :],
            dst_ref=out_ref.at[ccw_rows(ccw_sc), :],
            send_sem=ccw_ssem.at[k],
            recv_sem=ccw_rsem.at[k],
            device_id=(left,),
            device_id_type=pl.DeviceIdType.MESH,
        )
        cw.start()
        ccw.start()
        cw.wait()
        ccw.wait()


def kernel(ids, E):
    v_per, d = E.shape
    t = ids.shape[0]
    my_pos = lax.axis_index("i")

    local = ids - my_pos * v_per
    mask = (local >= 0) & (local < v_per)
    idx = jnp.clip(local, 0, v_per - 1).astype(jnp.int32)
    hit = mask.astype(jnp.int32)

    chunk = t // (2 * N_DEV)
    return pl.pallas_call(
        _body,
        out_shape=jax.ShapeDtypeStruct((t, d), jnp.float32),
        in_specs=[
            pl.BlockSpec(memory_space=pltpu.SMEM),
            pl.BlockSpec(memory_space=pltpu.SMEM),
            pl.BlockSpec(memory_space=pltpu.MemorySpace.HBM),
        ],
        out_specs=pl.BlockSpec(memory_space=pltpu.VMEM),
        scratch_shapes=[
            pltpu.VMEM((N_DEV - 1, chunk, d), jnp.float32),
            pltpu.VMEM((N_DEV - 1, chunk, d), jnp.float32),
            pltpu.SemaphoreType.DMA((K_DMA,)),
            pltpu.SemaphoreType.DMA((N_STEP,)),
            pltpu.SemaphoreType.DMA((N_STEP,)),
            pltpu.SemaphoreType.DMA((N_STEP,)),
            pltpu.SemaphoreType.DMA((N_STEP,)),
        ],
        compiler_params=pltpu.CompilerParams(collective_id=0),
    )(idx, hit, E)


# device time: 75175 ns/iter; 1.4039x vs baseline; 1.1493x over previous
import jax
import jax.numpy as jnp
from jax import lax
from jax.experimental import pallas as pl
from jax.experimental.pallas import tpu as pltpu

N_DEV = 4
N_STEP = 2 * (N_DEV - 1)
K_DMA = 32


def _body(pos_ref, idx_ref, nhit_ref, e_ref, out_ref, cw_buf, ccw_buf,
          gather_sems, cw_ssem, cw_rsem, ccw_ssem, ccw_rsem):
    t, d = out_ref.shape
    half = t // 2
    chunk = half // N_DEV

    my = lax.axis_index("i")
    left = (my - 1) % N_DEV
    right = (my + 1) % N_DEV

    def cw_rows(c):
        return pl.ds(c * chunk, chunk)

    def ccw_rows(c):
        return pl.ds(half + c * chunk, chunk)

    def row_dma(j):
        return pltpu.make_async_copy(
            e_ref.at[pl.ds(idx_ref[j], 1), :],
            out_ref.at[pl.ds(pos_ref[j], 1), :],
            gather_sems.at[j % K_DMA],
        )

    barrier_sem = pltpu.get_barrier_semaphore()
    for nbr in [left, right]:
        pl.semaphore_signal(
            barrier_sem, inc=1,
            device_id=(nbr,), device_id_type=pl.DeviceIdType.MESH,
        )
    pl.semaphore_wait(barrier_sem, 2)

    out_ref[:, :] = jnp.zeros((t, d), jnp.float32)

    n_hit = nhit_ref[0]

    def gather_step(j, carry):
        @pl.when(j >= K_DMA)
        def _():
            row_dma(j - K_DMA).wait()
        row_dma(j).start()
        return carry

    lax.fori_loop(0, n_hit, gather_step, 0)

    def drain_step(j, carry):
        row_dma(j).wait()
        return carry

    lax.fori_loop(jnp.maximum(n_hit - K_DMA, 0), n_hit, drain_step, 0)

    for s in range(N_DEV - 1):
        cw_sc = (my - s) % N_DEV
        cw_rc = (my - s - 1) % N_DEV
        ccw_sc = (my + s) % N_DEV
        ccw_rc = (my + s + 1) % N_DEV
        cw = pltpu.make_async_remote_copy(
            src_ref=out_ref.at[cw_rows(cw_sc), :],
            dst_ref=cw_buf.at[s],
            send_sem=cw_ssem.at[s],
            recv_sem=cw_rsem.at[s],
            device_id=(right,),
            device_id_type=pl.DeviceIdType.MESH,
        )
        ccw = pltpu.make_async_remote_copy(
            src_ref=out_ref.at[ccw_rows(ccw_sc), :],
            dst_ref=ccw_buf.at[s],
            send_sem=ccw_ssem.at[s],
            recv_sem=ccw_rsem.at[s],
            device_id=(left,),
            device_id_type=pl.DeviceIdType.MESH,
        )
        cw.start()
        ccw.start()
        cw.wait()
        ccw.wait()
        out_ref[cw_rows(cw_rc), :] = (
            out_ref[cw_rows(cw_rc), :] + cw_buf[s, :, :]
        )
        out_ref[ccw_rows(ccw_rc), :] = (
            out_ref[ccw_rows(ccw_rc), :] + ccw_buf[s, :, :]
        )

    for s in range(N_DEV - 1):
        k = (N_DEV - 1) + s
        cw_sc = (my + 1 - s) % N_DEV
        ccw_sc = (my - 1 + s) % N_DEV
        cw = pltpu.make_async_remote_copy(
            src_ref=out_ref.at[cw_rows(cw_sc), :],
            dst_ref=out_ref.at[cw_rows(cw_sc), :],
            send_sem=cw_ssem.at[k],
            recv_sem=cw_rsem.at[k],
            device_id=(right,),
            device_id_type=pl.DeviceIdType.MESH,
        )
        ccw = pltpu.make_async_remote_copy(
            src_ref=out_ref.at[ccw_rows(ccw_sc), :],
            dst_ref=out_ref.at[ccw_rows(ccw_sc), :],
            send_sem=ccw_ssem.at[k],
            recv_sem=ccw_rsem.at[k],
            device_id=(left,),
            device_id_type=pl.DeviceIdType.MESH,
        )
        cw.start()
        ccw.start()
        cw.wait()
        ccw.wait()


def kernel(ids, E):
    v_per, d = E.shape
    t = ids.shape[0]
    my_pos = lax.axis_index("i")

    local = ids - my_pos * v_per
    mask = (local >= 0) & (local < v_per)
    (pos,) = jnp.nonzero(mask, size=t, fill_value=0)
    pos = pos.astype(jnp.int32)
    idx = jnp.clip(local[pos], 0, v_per - 1).astype(jnp.int32)
    nhit = jnp.sum(mask.astype(jnp.int32)).reshape((1,))

    chunk = t // (2 * N_DEV)
    return pl.pallas_call(
        _body,
        out_shape=jax.ShapeDtypeStruct((t, d), jnp.float32),
        in_specs=[
            pl.BlockSpec(memory_space=pltpu.SMEM),
            pl.BlockSpec(memory_space=pltpu.SMEM),
            pl.BlockSpec(memory_space=pltpu.SMEM),
            pl.BlockSpec(memory_space=pltpu.MemorySpace.HBM),
        ],
        out_specs=pl.BlockSpec(memory_space=pltpu.VMEM),
        scratch_shapes=[
            pltpu.VMEM((N_DEV - 1, chunk, d), jnp.float32),
            pltpu.VMEM((N_DEV - 1, chunk, d), jnp.float32),
            pltpu.SemaphoreType.DMA((K_DMA,)),
            pltpu.SemaphoreType.DMA((N_STEP,)),
            pltpu.SemaphoreType.DMA((N_STEP,)),
            pltpu.SemaphoreType.DMA((N_STEP,)),
            pltpu.SemaphoreType.DMA((N_STEP,)),
        ],
        compiler_params=pltpu.CompilerParams(collective_id=0),
    )(pos, idx, nhit, E)


# device time: 62178 ns/iter; 1.6974x vs baseline; 1.2090x over previous
import jax
import jax.numpy as jnp
from jax import lax
from jax.experimental import pallas as pl
from jax.experimental.pallas import tpu as pltpu

N_DEV = 4
N_STEP = 2 * (N_DEV - 1)
K_DMA = 32


def _body(pos_ref, idx_ref, nhit_ref, e_ref, out_ref, cw_buf, ccw_buf,
          gather_sems, cw_ssem, cw_rsem, ccw_ssem, ccw_rsem):
    t, d = out_ref.shape
    half = t // 2
    chunk = half // N_DEV

    my = lax.axis_index("i")
    left = (my - 1) % N_DEV
    right = (my + 1) % N_DEV

    def cw_rows(c):
        return pl.ds(c * chunk, chunk)

    def ccw_rows(c):
        return pl.ds(half + c * chunk, chunk)

    def row_dma(j):
        return pltpu.make_async_copy(
            e_ref.at[pl.ds(idx_ref[j], 1), :],
            out_ref.at[pl.ds(pos_ref[j], 1), :],
            gather_sems.at[j % K_DMA],
        )

    barrier_sem = pltpu.get_barrier_semaphore()
    for nbr in [left, right]:
        pl.semaphore_signal(
            barrier_sem, inc=1,
            device_id=(nbr,), device_id_type=pl.DeviceIdType.MESH,
        )
    pl.semaphore_wait(barrier_sem, 2)

    out_ref[:, :] = jnp.zeros((t, d), jnp.float32)

    n_hit = nhit_ref[0]

    def gather_step(j, carry):
        @pl.when(j >= K_DMA)
        def _():
            row_dma(j - K_DMA).wait()
        row_dma(j).start()
        return carry

    lax.fori_loop(0, n_hit, gather_step, 0)

    def drain_step(j, carry):
        row_dma(j).wait()
        return carry

    lax.fori_loop(jnp.maximum(n_hit - K_DMA, 0), n_hit, drain_step, 0)

    for s in range(N_DEV - 1):
        cw_sc = (my - s) % N_DEV
        cw_rc = (my - s - 1) % N_DEV
        ccw_sc = (my + s) % N_DEV
        ccw_rc = (my + s + 1) % N_DEV
        cw = pltpu.make_async_remote_copy(
            src_ref=out_ref.at[cw_rows(cw_sc), :],
            dst_ref=cw_buf.at[s],
            send_sem=cw_ssem.at[s],
            recv_sem=cw_rsem.at[s],
            device_id=(right,),
            device_id_type=pl.DeviceIdType.MESH,
        )
        ccw = pltpu.make_async_remote_copy(
            src_ref=out_ref.at[ccw_rows(ccw_sc), :],
            dst_ref=ccw_buf.at[s],
            send_sem=ccw_ssem.at[s],
            recv_sem=ccw_rsem.at[s],
            device_id=(left,),
            device_id_type=pl.DeviceIdType.MESH,
        )
        cw.start()
        ccw.start()
        cw.wait()
        ccw.wait()
        out_ref[cw_rows(cw_rc), :] = (
            out_ref[cw_rows(cw_rc), :] + cw_buf[s, :, :]
        )
        out_ref[ccw_rows(ccw_rc), :] = (
            out_ref[ccw_rows(ccw_rc), :] + ccw_buf[s, :, :]
        )

    for s in range(N_DEV - 1):
        k = (N_DEV - 1) + s
        cw_sc = (my + 1 - s) % N_DEV
        ccw_sc = (my - 1 + s) % N_DEV
        cw = pltpu.make_async_remote_copy(
            src_ref=out_ref.at[cw_rows(cw_sc), :],
            dst_ref=out_ref.at[cw_rows(cw_sc), :],
            send_sem=cw_ssem.at[k],
            recv_sem=cw_rsem.at[k],
            device_id=(right,),
            device_id_type=pl.DeviceIdType.MESH,
        )
        ccw = pltpu.make_async_remote_copy(
            src_ref=out_ref.at[ccw_rows(ccw_sc), :],
            dst_ref=out_ref.at[ccw_rows(ccw_sc), :],
            send_sem=ccw_ssem.at[k],
            recv_sem=ccw_rsem.at[k],
            device_id=(left,),
            device_id_type=pl.DeviceIdType.MESH,
        )
        cw.start()
        ccw.start()
        cw.wait()
        ccw.wait()


def kernel(ids, E):
    v_per, d = E.shape
    t = ids.shape[0]
    my_pos = lax.axis_index("i")

    local = ids - my_pos * v_per
    mask = (local >= 0) & (local < v_per)
    c = jnp.cumsum(mask.astype(jnp.int32))
    slots = jnp.arange(t, dtype=jnp.int32)
    m = ((c[:, None] == slots[None, :] + 1) & mask[:, None]).astype(jnp.int32)
    pos = jnp.sum(m * slots[:, None], axis=0).astype(jnp.int32)
    idx = jnp.clip(jnp.sum(m * local[:, None], axis=0),
                   0, v_per - 1).astype(jnp.int32)
    nhit = c[-1:]

    chunk = t // (2 * N_DEV)
    return pl.pallas_call(
        _body,
        out_shape=jax.ShapeDtypeStruct((t, d), jnp.float32),
        in_specs=[
            pl.BlockSpec(memory_space=pltpu.SMEM),
            pl.BlockSpec(memory_space=pltpu.SMEM),
            pl.BlockSpec(memory_space=pltpu.SMEM),
            pl.BlockSpec(memory_space=pltpu.MemorySpace.HBM),
        ],
        out_specs=pl.BlockSpec(memory_space=pltpu.VMEM),
        scratch_shapes=[
            pltpu.VMEM((N_DEV - 1, chunk, d), jnp.float32),
            pltpu.VMEM((N_DEV - 1, chunk, d), jnp.float32),
            pltpu.SemaphoreType.DMA((K_DMA,)),
            pltpu.SemaphoreType.DMA((N_STEP,)),
            pltpu.SemaphoreType.DMA((N_STEP,)),
            pltpu.SemaphoreType.DMA((N_STEP,)),
            pltpu.SemaphoreType.DMA((N_STEP,)),
        ],
        compiler_params=pltpu.CompilerParams(collective_id=0),
    )(pos, idx, nhit, E)


# device time: 56786 ns/iter; 1.8586x vs baseline; 1.0950x over previous
import jax
import jax.numpy as jnp
from jax import lax
from jax.experimental import pallas as pl
from jax.experimental.pallas import tpu as pltpu

N_DEV = 4
K_DMA = 32


def _body(pos_ref, idx_ref, cnt_ref, e_ref, out_ref,
          bx1, bx2, by1, by2, gather_sems, xs, xr, ys, yr):
    t, d = out_ref.shape
    t2 = t // 2
    b2 = t // 4
    b4 = t // 8

    my = lax.axis_index("i")
    a = my % 2
    b = my // 2
    k1 = (a + b) % 2
    p_a = my + 1 - 2 * a
    p_b = 3 - my

    def xfer(src_rows, n_rows, dst, ssem, rsem, peer):
        return pltpu.make_async_remote_copy(
            src_ref=out_ref.at[pl.ds(src_rows, n_rows), :],
            dst_ref=dst,
            send_sem=ssem,
            recv_sem=rsem,
            device_id=(peer,),
            device_id_type=pl.DeviceIdType.MESH,
        )

    def row_dma(j):
        return pltpu.make_async_copy(
            e_ref.at[pl.ds(idx_ref[j], 1), :],
            out_ref.at[pl.ds(pos_ref[j], 1), :],
            gather_sems.at[j % K_DMA],
        )

    def gather(lo, hi):
        def step(j, carry):
            @pl.when(j - lo >= K_DMA)
            def _():
                row_dma(j - K_DMA).wait()
            row_dma(j).start()
            return carry

        lax.fori_loop(lo, hi, step, 0)

        def drain(j, carry):
            row_dma(j).wait()
            return carry

        lax.fori_loop(jnp.maximum(hi - K_DMA, lo), hi, drain, 0)

    barrier_sem = pltpu.get_barrier_semaphore()
    for nbr in [p_a, p_b]:
        pl.semaphore_signal(
            barrier_sem, inc=1,
            device_id=(nbr,), device_id_type=pl.DeviceIdType.MESH,
        )
    pl.semaphore_wait(barrier_sem, 2)

    out_ref[:, :] = jnp.zeros((t, d), jnp.float32)

    n1 = cnt_ref[0]
    n = cnt_ref[1]

    gather(0, n1)

    x_send = (1 - k1) * b2
    y_send = t2 + (1 - b) * b2
    x1 = xfer(x_send, b2, bx1, xs.at[0], xr.at[0], p_a)
    y1 = xfer(y_send, b2, by1, ys.at[0], yr.at[0], p_b)
    x1.start()
    y1.start()

    gather(n1, n)

    x_keep = k1 * b2
    y_keep = t2 + b * b2
    x1.wait()
    out_ref[pl.ds(x_keep, b2), :] = (
        out_ref[pl.ds(x_keep, b2), :] + bx1[:, :]
    )
    x_q_keep = x_keep + b * b4
    x2 = xfer(x_keep + (1 - b) * b4, b4, bx2, xs.at[1], xr.at[1], p_b)
    x2.start()
    y1.wait()
    out_ref[pl.ds(y_keep, b2), :] = (
        out_ref[pl.ds(y_keep, b2), :] + by1[:, :]
    )
    y_q_keep = y_keep + a * b4
    y2 = xfer(y_keep + (1 - a) * b4, b4, by2, ys.at[1], yr.at[1], p_a)
    y2.start()

    x2.wait()
    out_ref[pl.ds(x_q_keep, b4), :] = (
        out_ref[pl.ds(x_q_keep, b4), :] + bx2[:, :]
    )
    y2.wait()
    out_ref[pl.ds(y_q_keep, b4), :] = (
        out_ref[pl.ds(y_q_keep, b4), :] + by2[:, :]
    )

    x3 = xfer(x_q_keep, b4, out_ref.at[pl.ds(x_q_keep, b4), :],
              xs.at[2], xr.at[2], p_b)
    y3 = xfer(y_q_keep, b4, out_ref.at[pl.ds(y_q_keep, b4), :],
              ys.at[2], yr.at[2], p_a)
    x3.start()
    y3.start()
    x3.wait()
    y3.wait()

    x4 = xfer(x_keep, b2, out_ref.at[pl.ds(x_keep, b2), :],
              xs.at[3], xr.at[3], p_a)
    y4 = xfer(y_keep, b2, out_ref.at[pl.ds(y_keep, b2), :],
              ys.at[3], yr.at[3], p_b)
    x4.start()
    y4.start()
    x4.wait()
    y4.wait()


def kernel(ids, E):
    v_per, d = E.shape
    t = ids.shape[0]
    my_pos = lax.axis_index("i")

    local = ids - my_pos * v_per
    mask = (local >= 0) & (local < v_per)

    a = my_pos % 2
    b = my_pos // 2
    k1 = (a + b) % 2
    b2 = t // 4
    x_send = (1 - k1) * b2
    y_send = t // 2 + (1 - b) * b2
    rows = jnp.arange(t, dtype=jnp.int32)
    in_send = ((rows >= x_send) & (rows < x_send + b2)) | (
        (rows >= y_send) & (rows < y_send + b2)
    )

    cs = jnp.cumsum((mask & in_send).astype(jnp.int32))
    co = jnp.cumsum((mask & ~in_send).astype(jnp.int32))
    n1 = cs[-1]
    slot = jnp.where(in_send, cs, n1 + co) - 1
    slots = jnp.arange(t, dtype=jnp.int32)
    m = ((slot[:, None] == slots[None, :]) & mask[:, None]).astype(jnp.int32)
    pos = jnp.sum(m * rows[:, None], axis=0).astype(jnp.int32)
    idx = jnp.clip(jnp.sum(m * local[:, None], axis=0),
                   0, v_per - 1).astype(jnp.int32)
    cnt = jnp.stack([n1, n1 + co[-1]]).astype(jnp.int32)

    return pl.pallas_call(
        _body,
        out_shape=jax.ShapeDtypeStruct((t, d), jnp.float32),
        in_specs=[
            pl.BlockSpec(memory_space=pltpu.SMEM),
            pl.BlockSpec(memory_space=pltpu.SMEM),
            pl.BlockSpec(memory_space=pltpu.SMEM),
            pl.BlockSpec(memory_space=pltpu.MemorySpace.HBM),
        ],
        out_specs=pl.BlockSpec(memory_space=pltpu.VMEM),
        scratch_shapes=[
            pltpu.VMEM((t // 4, d), jnp.float32),
            pltpu.VMEM((t // 8, d), jnp.float32),
            pltpu.VMEM((t // 4, d), jnp.float32),
            pltpu.VMEM((t // 8, d), jnp.float32),
            pltpu.SemaphoreType.DMA((K_DMA,)),
            pltpu.SemaphoreType.DMA((4,)),
            pltpu.SemaphoreType.DMA((4,)),
            pltpu.SemaphoreType.DMA((4,)),
            pltpu.SemaphoreType.DMA((4,)),
        ],
        compiler_params=pltpu.CompilerParams(collective_id=0),
    )(pos, idx, cnt, E)
